# Initial kernel scaffold; baseline (speedup 1.0000x reference)
import jax
import jax.numpy as jnp
from jax.experimental import pallas as pl


def kernel(x, vocab_vectors, W, b):
    raise NotImplementedError("write your pallas kernel here")


# speedup vs baseline: 7.0434x; 7.0434x over previous
import functools

import jax
import jax.numpy as jnp
from jax import lax
from jax.experimental import pallas as pl
from jax.experimental.pallas import tpu as pltpu
from jax.experimental.pallas import tpu_sc as plsc

_NC = 2
_NS = 16
_LANES = 16
_NW = _NC * _NS


def _matmul_stage(table, wt, v, d, lp):
    rb = 2000

    def body(t_ref, w_ref, p_ref):
        p_ref[...] = jnp.dot(t_ref[...], w_ref[...],
                             preferred_element_type=jnp.float32)

    return pl.pallas_call(
        body,
        grid=(v // rb,),
        in_specs=[
            pl.BlockSpec((rb, d), lambda i: (i, 0)),
            pl.BlockSpec((d, lp), lambda i: (0, 0)),
        ],
        out_specs=pl.BlockSpec((rb, lp), lambda i: (i, 0)),
        out_shape=jax.ShapeDtypeStruct((v, lp), jnp.float32),
    )(table, wt)


def _gather_sum_stage(xt, p_flat, batch, seq, lp):
    bpw = batch // _NW
    jg = bpw // _LANES

    mesh = plsc.VectorSubcoreMesh(core_axis_name="c", subcore_axis_name="s")

    @functools.partial(
        pl.kernel,
        out_type=jax.ShapeDtypeStruct((batch,), jnp.float32),
        mesh=mesh,
        scratch_types=[
            pltpu.VMEM((seq, bpw), jnp.int32),
            pltpu.VMEM((seq, bpw), jnp.int32),
            pltpu.VMEM((seq, bpw), jnp.float32),
            pltpu.VMEM((bpw,), jnp.float32),
            pltpu.SemaphoreType.DMA,
        ],
    )
    def sc_kernel(xt_hbm, p_hbm, out_hbm, xv, idxv, gv, acc, sem):
        wid = lax.axis_index("s") * _NC + lax.axis_index("c")
        base = wid * bpw
        pltpu.sync_copy(xt_hbm.at[:, pl.ds(base, bpw)], xv)

        def build(l, carry):
            for j in range(jg):
                idxv[l, pl.ds(j * _LANES, _LANES)] = (
                    xv[l, pl.ds(j * _LANES, _LANES)] * lp + l)
            return carry

        lax.fori_loop(0, seq, build, 0)

        def fire(l, carry):
            pltpu.async_copy(p_hbm.at[idxv.at[l]], gv.at[l], sem)
            return carry

        lax.fori_loop(0, seq, fire, 0)

        def drain(l, carry):
            pltpu.make_async_copy(p_hbm.at[idxv.at[l]], gv.at[l], sem).wait()
            return carry

        lax.fori_loop(0, seq, drain, 0)

        zeros = jnp.zeros((_LANES,), jnp.float32)
        for j in range(jg):
            acc[pl.ds(j * _LANES, _LANES)] = zeros

        def accum(l, carry):
            for j in range(jg):
                plsc.addupdate(acc.at[pl.ds(j * _LANES, _LANES)],
                               gv[l, pl.ds(j * _LANES, _LANES)])
            return carry

        lax.fori_loop(0, seq, accum, 0)

        pltpu.sync_copy(acc, out_hbm.at[pl.ds(base, bpw)])

    return sc_kernel(xt, p_flat)


def kernel(x, vocab_vectors, W, b):
    batch, seq = x.shape
    v, d = vocab_vectors.shape
    lp = 64

    x = x.astype(jnp.int32)
    wt = W[:, 0].reshape(seq, d).T
    wt = jnp.pad(wt, ((0, 0), (0, lp - seq)))

    p = _matmul_stage(vocab_vectors, wt, v, d, lp)
    out = _gather_sum_stage(x.T, p.reshape(v * lp), batch, seq, lp)
    return out.reshape(batch, 1) + b[0]


# speedup vs baseline: 10.2234x; 1.4515x over previous
import functools

import jax
import jax.numpy as jnp
from jax import lax
from jax.experimental import pallas as pl
from jax.experimental.pallas import tpu as pltpu
from jax.experimental.pallas import tpu_sc as plsc

_NC = 2
_NS = 16
_LANES = 16
_NW = _NC * _NS


def _matmul_stage(table, wt, v, d, lp):
    rb = 2000

    def body(t_ref, w_ref, p_ref):
        p_ref[...] = jnp.dot(t_ref[...], w_ref[...],
                             preferred_element_type=jnp.float32)

    return pl.pallas_call(
        body,
        grid=(v // rb,),
        in_specs=[
            pl.BlockSpec((rb, d), lambda i: (i, 0)),
            pl.BlockSpec((d, lp), lambda i: (0, 0)),
        ],
        out_specs=pl.BlockSpec((rb, lp), lambda i: (i, 0)),
        out_shape=jax.ShapeDtypeStruct((v, lp), jnp.float32),
    )(table, wt)


def _gather_sum_stage(xt, p_flat, batch, seq, lp):
    bpw = batch // _NW
    jg = bpw // _LANES

    mesh = plsc.VectorSubcoreMesh(core_axis_name="c", subcore_axis_name="s")

    @functools.partial(
        pl.kernel,
        out_type=jax.ShapeDtypeStruct((batch,), jnp.float32),
        mesh=mesh,
        scratch_types=[
            pltpu.VMEM((seq, bpw), jnp.int32),
            pltpu.VMEM((seq, bpw), jnp.int32),
            pltpu.VMEM((seq, bpw), jnp.float32),
            pltpu.VMEM((bpw,), jnp.float32),
            pltpu.SemaphoreType.DMA,
        ],
    )
    def sc_kernel(xt_hbm, p_hbm, out_hbm, xv, idxv, gv, acc, sem):
        wid = lax.axis_index("s") * _NC + lax.axis_index("c")
        base = wid * bpw
        pltpu.sync_copy(xt_hbm.at[:, pl.ds(base, bpw)], xv)

        def build(l, carry):
            for j in range(jg):
                idxv[l, pl.ds(j * _LANES, _LANES)] = (
                    xv[l, pl.ds(j * _LANES, _LANES)] * lp + l)
            return carry

        lax.fori_loop(0, seq, build, 0)

        def fire(l, carry):
            pltpu.async_copy(p_hbm.at[idxv.at[l]], gv.at[l], sem)
            return carry

        lax.fori_loop(0, seq, fire, 0)

        def drain(l, carry):
            pltpu.make_async_copy(p_hbm.at[idxv.at[l]], gv.at[l], sem).wait()
            return carry

        lax.fori_loop(0, seq, drain, 0)

        zeros = jnp.zeros((_LANES,), jnp.float32)
        for j in range(jg):
            acc[pl.ds(j * _LANES, _LANES)] = zeros

        def accum(l, carry):
            for j in range(jg):
                plsc.addupdate(acc.at[pl.ds(j * _LANES, _LANES)],
                               gv[l, pl.ds(j * _LANES, _LANES)])
            return carry

        lax.fori_loop(0, seq, accum, 0)

        pltpu.sync_copy(acc, out_hbm.at[pl.ds(base, bpw)])

    return sc_kernel(xt, p_flat)


def kernel(x, vocab_vectors, W, b):
    batch, seq = x.shape
    v, d = vocab_vectors.shape
    lp = 128

    x = x.astype(jnp.int32)
    wt = W[:, 0].reshape(seq, d).T
    wt = jnp.pad(wt, ((0, 0), (0, lp - seq)))

    p = _matmul_stage(vocab_vectors, wt, v, d, lp)
    out = _gather_sum_stage(x.T, p.reshape(v * lp), batch, seq, lp)
    return out.reshape(batch, 1) + b[0]


# speedup vs baseline: 12.5671x; 1.2292x over previous
import functools

import jax
import jax.numpy as jnp
from jax import lax
from jax.experimental import pallas as pl
from jax.experimental.pallas import tpu as pltpu
from jax.experimental.pallas import tpu_sc as plsc

_NC = 2
_NS = 16
_LANES = 16
_NW = _NC * _NS


def _matmul_stage(table, wt, v, d, lp):
    rb = 4000

    def body(t_ref, w_ref, p_ref):
        p_ref[...] = jnp.dot(t_ref[...], w_ref[...],
                             preferred_element_type=jnp.float32)

    return pl.pallas_call(
        body,
        grid=(v // rb,),
        in_specs=[
            pl.BlockSpec((rb, d), lambda i: (i, 0)),
            pl.BlockSpec((d, lp), lambda i: (0, 0)),
        ],
        out_specs=pl.BlockSpec((rb, lp), lambda i: (i, 0)),
        out_shape=jax.ShapeDtypeStruct((v, lp), jnp.float32),
    )(table, wt)


def _gather_sum_stage(xt, p_flat, batch, seq, lp):
    bpw = batch // _NW
    jg = bpw // _LANES

    mesh = plsc.VectorSubcoreMesh(core_axis_name="c", subcore_axis_name="s")

    @functools.partial(
        pl.kernel,
        out_type=jax.ShapeDtypeStruct((batch,), jnp.float32),
        mesh=mesh,
        scratch_types=[
            pltpu.VMEM((seq, bpw), jnp.int32),
            pltpu.VMEM((seq, bpw), jnp.int32),
            pltpu.VMEM((seq, bpw), jnp.float32),
            pltpu.VMEM((bpw,), jnp.float32),
            pltpu.SemaphoreType.DMA,
        ],
    )
    def sc_kernel(xt_hbm, p_hbm, out_hbm, xv, idxv, gv, acc, sem):
        wid = lax.axis_index("s") * _NC + lax.axis_index("c")
        base = wid * bpw
        pltpu.sync_copy(xt_hbm.at[:, pl.ds(base, bpw)], xv)

        def build(l, carry):
            for j in range(jg):
                idxv[l, pl.ds(j * _LANES, _LANES)] = (
                    xv[l, pl.ds(j * _LANES, _LANES)] * lp + l)
            return carry

        lax.fori_loop(0, seq, build, 0)

        def fire(l, carry):
            pltpu.async_copy(p_hbm.at[idxv.at[l]], gv.at[l], sem)
            return carry

        lax.fori_loop(0, seq, fire, 0)

        def drain(l, carry):
            pltpu.make_async_copy(p_hbm.at[idxv.at[l]], gv.at[l], sem).wait()
            return carry

        lax.fori_loop(0, seq, drain, 0)

        zeros = jnp.zeros((_LANES,), jnp.float32)
        for j in range(jg):
            acc[pl.ds(j * _LANES, _LANES)] = zeros

        def accum(l, carry):
            for j in range(jg):
                plsc.addupdate(acc.at[pl.ds(j * _LANES, _LANES)],
                               gv[l, pl.ds(j * _LANES, _LANES)])
            return carry

        lax.fori_loop(0, seq, accum, 0)

        pltpu.sync_copy(acc, out_hbm.at[pl.ds(base, bpw)])

    return sc_kernel(xt, p_flat)


def kernel(x, vocab_vectors, W, b):
    batch, seq = x.shape
    v, d = vocab_vectors.shape
    lp = 128

    x = x.astype(jnp.int32)
    wt = W[:, 0].reshape(seq, d).T
    wt = jnp.pad(wt, ((0, 0), (0, lp - seq)))

    p = _matmul_stage(vocab_vectors, wt, v, d, lp)
    out = _gather_sum_stage(x.T, p.reshape(v * lp), batch, seq, lp)
    return out.reshape(batch, 1) + b[0]


# speedup vs baseline: 13.6509x; 1.0862x over previous
import functools

import jax
import jax.numpy as jnp
from jax import lax
from jax.experimental import pallas as pl
from jax.experimental.pallas import tpu as pltpu
from jax.experimental.pallas import tpu_sc as plsc

_NC = 2
_NS = 16
_LANES = 16
_NW = _NC * _NS


def _matmul_stage(table, wt, v, d, lp):
    rb = 10000

    def body(t_ref, w_ref, p_ref):
        p_ref[...] = jnp.dot(t_ref[...], w_ref[...],
                             preferred_element_type=jnp.float32)

    return pl.pallas_call(
        body,
        grid=(v // rb,),
        in_specs=[
            pl.BlockSpec((rb, d), lambda i: (i, 0)),
            pl.BlockSpec((d, lp), lambda i: (0, 0)),
        ],
        out_specs=pl.BlockSpec((rb, lp), lambda i: (i, 0)),
        out_shape=jax.ShapeDtypeStruct((v, lp), jnp.float32),
    )(table, wt)


def _gather_sum_stage(xt, p_flat, batch, seq, lp):
    bpw = batch // _NW
    jg = bpw // _LANES

    mesh = plsc.VectorSubcoreMesh(core_axis_name="c", subcore_axis_name="s")

    @functools.partial(
        pl.kernel,
        out_type=jax.ShapeDtypeStruct((batch,), jnp.float32),
        mesh=mesh,
        scratch_types=[
            pltpu.VMEM((seq, bpw), jnp.int32),
            pltpu.VMEM((seq, bpw), jnp.int32),
            pltpu.VMEM((seq, bpw), jnp.float32),
            pltpu.VMEM((bpw,), jnp.float32),
            pltpu.SemaphoreType.DMA,
        ],
    )
    def sc_kernel(xt_hbm, p_hbm, out_hbm, xv, idxv, gv, acc, sem):
        wid = lax.axis_index("s") * _NC + lax.axis_index("c")
        base = wid * bpw
        pltpu.sync_copy(xt_hbm.at[:, pl.ds(base, bpw)], xv)

        def build(l, carry):
            for j in range(jg):
                idxv[l, pl.ds(j * _LANES, _LANES)] = (
                    xv[l, pl.ds(j * _LANES, _LANES)] * lp + l)
            return carry

        lax.fori_loop(0, seq, build, 0)

        def fire(l, carry):
            pltpu.async_copy(p_hbm.at[idxv.at[l]], gv.at[l], sem)
            return carry

        lax.fori_loop(0, seq, fire, 0)

        def drain(l, carry):
            pltpu.make_async_copy(p_hbm.at[idxv.at[l]], gv.at[l], sem).wait()
            return carry

        lax.fori_loop(0, seq, drain, 0)

        zeros = jnp.zeros((_LANES,), jnp.float32)
        for j in range(jg):
            acc[pl.ds(j * _LANES, _LANES)] = zeros

        def accum(l, carry):
            for j in range(jg):
                plsc.addupdate(acc.at[pl.ds(j * _LANES, _LANES)],
                               gv[l, pl.ds(j * _LANES, _LANES)])
            return carry

        lax.fori_loop(0, seq, accum, 0)

        pltpu.sync_copy(acc, out_hbm.at[pl.ds(base, bpw)])

    return sc_kernel(xt, p_flat)


def kernel(x, vocab_vectors, W, b):
    batch, seq = x.shape
    v, d = vocab_vectors.shape
    lp = 128

    x = x.astype(jnp.int32)
    wt = W[:, 0].reshape(seq, d).T
    wt = jnp.pad(wt, ((0, 0), (0, lp - seq)))

    p = _matmul_stage(vocab_vectors, wt, v, d, lp)
    out = _gather_sum_stage(x.T, p.reshape(v * lp), batch, seq, lp)
    return out.reshape(batch, 1) + b[0]


# speedup vs baseline: 13.9548x; 1.0223x over previous
import functools

import jax
import jax.numpy as jnp
from jax import lax
from jax.experimental import pallas as pl
from jax.experimental.pallas import tpu as pltpu
from jax.experimental.pallas import tpu_sc as plsc

_NC = 2
_NS = 16
_LANES = 16
_NW = _NC * _NS


def _matmul_stage(table, wt, v, d, lp):
    rb = 20000

    def body(t_ref, w_ref, p_ref):
        p_ref[...] = jnp.dot(t_ref[...], w_ref[...],
                             preferred_element_type=jnp.float32)

    return pl.pallas_call(
        body,
        grid=(v // rb,),
        in_specs=[
            pl.BlockSpec((rb, d), lambda i: (i, 0)),
            pl.BlockSpec((d, lp), lambda i: (0, 0)),
        ],
        out_specs=pl.BlockSpec((rb, lp), lambda i: (i, 0)),
        out_shape=jax.ShapeDtypeStruct((v, lp), jnp.float32),
    )(table, wt)


def _gather_sum_stage(xt, p_flat, batch, seq, lp):
    bpw = batch // _NW
    jg = bpw // _LANES

    mesh = plsc.VectorSubcoreMesh(core_axis_name="c", subcore_axis_name="s")

    @functools.partial(
        pl.kernel,
        out_type=jax.ShapeDtypeStruct((batch,), jnp.float32),
        mesh=mesh,
        scratch_types=[
            pltpu.VMEM((seq, bpw), jnp.int32),
            pltpu.VMEM((seq, bpw), jnp.int32),
            pltpu.VMEM((seq, bpw), jnp.float32),
            pltpu.VMEM((bpw,), jnp.float32),
            pltpu.SemaphoreType.DMA,
        ],
    )
    def sc_kernel(xt_hbm, p_hbm, out_hbm, xv, idxv, gv, acc, sem):
        wid = lax.axis_index("s") * _NC + lax.axis_index("c")
        base = wid * bpw
        pltpu.sync_copy(xt_hbm.at[:, pl.ds(base, bpw)], xv)

        def build(l, carry):
            for j in range(jg):
                idxv[l, pl.ds(j * _LANES, _LANES)] = (
                    xv[l, pl.ds(j * _LANES, _LANES)] * lp + l)
            return carry

        lax.fori_loop(0, seq, build, 0)

        def fire(l, carry):
            pltpu.async_copy(p_hbm.at[idxv.at[l]], gv.at[l], sem)
            return carry

        lax.fori_loop(0, seq, fire, 0)

        def drain(l, carry):
            pltpu.make_async_copy(p_hbm.at[idxv.at[l]], gv.at[l], sem).wait()
            return carry

        lax.fori_loop(0, seq, drain, 0)

        zeros = jnp.zeros((_LANES,), jnp.float32)
        for j in range(jg):
            acc[pl.ds(j * _LANES, _LANES)] = zeros

        def accum(l, carry):
            for j in range(jg):
                plsc.addupdate(acc.at[pl.ds(j * _LANES, _LANES)],
                               gv[l, pl.ds(j * _LANES, _LANES)])
            return carry

        lax.fori_loop(0, seq, accum, 0)

        pltpu.sync_copy(acc, out_hbm.at[pl.ds(base, bpw)])

    return sc_kernel(xt, p_flat)


def kernel(x, vocab_vectors, W, b):
    batch, seq = x.shape
    v, d = vocab_vectors.shape
    lp = 128

    x = x.astype(jnp.int32)
    wt = W[:, 0].reshape(seq, d).T
    wt = jnp.pad(wt, ((0, 0), (0, lp - seq)))

    p = _matmul_stage(vocab_vectors, wt, v, d, lp)
    out = _gather_sum_stage(x.T, p.reshape(v * lp), batch, seq, lp)
    return out.reshape(batch, 1) + b[0]
